# baseline (speedup 1.0000x reference)
import functools

import jax
import jax.numpy as jnp
from jax import lax
from jax.experimental import pallas as pl
from jax.experimental.pallas import tpu as pltpu
from jax.experimental.pallas import tpu_sc as plsc

NEG_SLOPE = 0.2
NC = 2
NS = 16
NW = NC * NS
L = 16


def _prologue_body(npad, x_ref, w_ref, av_ref, tab_ref):
    x = x_ref[...]
    w = w_ref[...]
    av = av_ref[...]
    wu = lax.dot(av[:, 0:2], w, preferred_element_type=jnp.float32)
    wv = lax.dot(av[:, 2:4], w, preferred_element_type=jnp.float32)
    g = jnp.concatenate([w, wu, wv], axis=0)
    bigt = lax.dot_general(g, x, (((1,), (1,)), ((), ())),
                           preferred_element_type=jnp.float32)
    h_dim = w.shape[0]
    heads = av.shape[0]
    n = x.shape[0]
    tab_ref[0:h_dim + heads, 0:n] = bigt[0:h_dim + heads, :]
    us = bigt[h_dim:h_dim + heads, :]
    vd = bigt[h_dim + heads:h_dim + 2 * heads, :]
    bound = jnp.max(us, axis=1) + jnp.max(vd, axis=1)
    m = jnp.max(jnp.maximum(bound, NEG_SLOPE * bound))
    aux_row = h_dim + heads
    tab_ref[aux_row, 0:heads] = av[:, 2]
    tab_ref[aux_row, heads:2 * heads] = av[:, 3]
    tab_ref[aux_row, 2 * heads:2 * heads + L] = jnp.full((L,), m,
                                                         dtype=jnp.float32)


def _prologue(x, w, av):
    n = x.shape[0]
    npad = ((n + 127) // 128) * 128
    return pl.pallas_call(
        functools.partial(_prologue_body, npad),
        out_shape=jax.ShapeDtypeStruct((16, npad), jnp.float32),
    )(x, w, av)


def _sc_body(heads, epw, unroll, e_total, npad, *refs):
    hg_heads = heads // 2
    epad = e_total
    esd_ref, tab_ref, out_ref = refs[0:3]
    scr = refs[3:]
    es_v, ed_v, h0_v, h1_v = scr[0:4]
    us_vs = scr[4:4 + hg_heads]
    aux_v, o_v, sem = scr[4 + hg_heads:7 + hg_heads]

    cid = lax.axis_index("c")
    sid = lax.axis_index("s")
    wid = sid * NC + cid
    hg = wid % 2
    esl = wid // 2

    copies = [
        pltpu.async_copy(esd_ref.at[pl.ds(esl * epw, epw)], es_v, sem),
        pltpu.async_copy(esd_ref.at[pl.ds(epad + esl * epw, epw)], ed_v, sem),
        pltpu.async_copy(tab_ref.at[0], h0_v, sem),
        pltpu.async_copy(tab_ref.at[1], h1_v, sem),
        pltpu.async_copy(tab_ref.at[2 + heads, pl.ds(0, 4 * L)], aux_v, sem),
    ]

    @pl.when(hg == 0)
    def _():
        cs = [pltpu.async_copy(tab_ref.at[2 + h], us_vs[h], sem)
              for h in range(hg_heads)]
        for c in cs:
            c.wait()

    @pl.when(hg == 1)
    def _():
        cs = [pltpu.async_copy(tab_ref.at[2 + hg_heads + h], us_vs[h], sem)
              for h in range(hg_heads)]
        for c in cs:
            c.wait()

    for c in copies:
        c.wait()

    m_vec = aux_v[pl.ds(2 * heads, L)]
    hb = hg * hg_heads
    av2 = [plsc.load_gather(aux_v, [jnp.full((L,), h, jnp.int32) + hb])
           for h in range(hg_heads)]
    av3 = [plsc.load_gather(aux_v, [jnp.full((L,), heads + h, jnp.int32) + hb])
           for h in range(hg_heads)]

    init = tuple(jnp.zeros((L,), jnp.float32) for _ in range(3 * hg_heads))

    @plsc.parallel_loop(0, epw // L, 1, unroll=unroll, carry=init)
    def accs(i, accs):
        accs = list(accs)
        s = es_v[pl.ds(i * L, L)]
        d = ed_v[pl.ds(i * L, L)]
        hd0 = plsc.load_gather(h0_v, [d])
        hd1 = plsc.load_gather(h1_v, [d])
        for h in range(hg_heads):
            us_h = plsc.load_gather(us_vs[h], [s])
            z = us_h + av2[h] * hd0 + av3[h] * hd1
            y = jnp.maximum(z, NEG_SLOPE * z)
            p = jnp.exp(y - m_vec)
            accs[3 * h] = accs[3 * h] + p
            accs[3 * h + 1] = accs[3 * h + 1] + p * hd0
            accs[3 * h + 2] = accs[3 * h + 2] + p * hd1
        return tuple(accs)

    for h in range(hg_heads):
        o_v[pl.ds((0 * hg_heads + h) * L, L)] = accs[3 * h]
        o_v[pl.ds((1 * hg_heads + h) * L, L)] = accs[3 * h + 1]
        o_v[pl.ds((2 * hg_heads + h) * L, L)] = accs[3 * h + 2]
    nacc = 3 * hg_heads * L
    pltpu.sync_copy(o_v, out_ref.at[pl.ds(wid * nacc, nacc)])


def _sc_main(esd, tab, heads):
    e_total = esd.shape[0] // 2
    epw = e_total // (NW // 2)
    npad = tab.shape[1]
    unroll = 5
    assert epw % (L * unroll) == 0
    mesh = plsc.VectorSubcoreMesh(core_axis_name="c", subcore_axis_name="s",
                                  num_cores=NC, num_subcores=NS)
    nacc = 3 * (heads // 2) * L
    f = pl.kernel(
        functools.partial(_sc_body, heads, epw, unroll, e_total, npad),
        out_type=jax.ShapeDtypeStruct((NW * nacc,), jnp.float32),
        mesh=mesh,
        compiler_params=pltpu.CompilerParams(needs_layout_passes=False),
        scratch_types=(
            [pltpu.VMEM((epw,), jnp.int32)] * 2
            + [pltpu.VMEM((npad,), jnp.float32)] * (2 + heads // 2)
            + [pltpu.VMEM((4 * L,), jnp.float32),
               pltpu.VMEM((nacc,), jnp.float32),
               pltpu.SemaphoreType.DMA]
        ),
    )
    return f(esd, tab)


def _epilogue_body(heads, p_ref, o_ref):
    hh = heads // 2
    t = jnp.sum(jnp.sum(p_ref[...], axis=-1), axis=0)
    s = jnp.concatenate([t[0, 0:hh], t[1, 0:hh]])
    t0 = jnp.concatenate([t[0, hh:2 * hh], t[1, hh:2 * hh]])
    t1 = jnp.concatenate([t[0, 2 * hh:3 * hh], t[1, 2 * hh:3 * hh]])
    o_ref[...] = jax.nn.sigmoid(jnp.stack([t0 / s, t1 / s], axis=0))


def _epilogue(partials, heads):
    return pl.pallas_call(
        functools.partial(_epilogue_body, heads),
        out_shape=jax.ShapeDtypeStruct((2, heads), jnp.float32),
    )(partials)


def kernel(x, edges, a, W, attention_vectors):
    del a
    e = edges.shape[0]
    heads = attention_vectors.shape[0]
    assert e % (NW * L) == 0

    tab = _prologue(x, W, attention_vectors)
    esd = edges.astype(jnp.int32).T.reshape(-1)
    partials = _sc_main(esd, tab, heads)
    out28 = _epilogue(
        partials.reshape(NW // 2, 2, 3 * (heads // 2), L), heads)
    return out28.T


# speedup vs baseline: 1.0558x; 1.0558x over previous
import functools

import jax
import jax.numpy as jnp
from jax import lax
from jax.experimental import pallas as pl
from jax.experimental.pallas import tpu as pltpu
from jax.experimental.pallas import tpu_sc as plsc

NEG_SLOPE = 0.2
NC = 2
NS = 16
NW = NC * NS
L = 16


def _prologue_body(npad, x_ref, w_ref, av_ref, tab_ref):
    x = x_ref[...]
    w = w_ref[...]
    av = av_ref[...]
    wu = lax.dot(av[:, 0:2], w, preferred_element_type=jnp.float32)
    wv = lax.dot(av[:, 2:4], w, preferred_element_type=jnp.float32)
    g = jnp.concatenate([w, wu, wv], axis=0)
    bigt = lax.dot_general(g, x, (((1,), (1,)), ((), ())),
                           preferred_element_type=jnp.float32)
    h_dim = w.shape[0]
    heads = av.shape[0]
    n = x.shape[0]
    tab_ref[0:h_dim + heads, 0:n] = bigt[0:h_dim + heads, :]
    us = bigt[h_dim:h_dim + heads, :]
    vd = bigt[h_dim + heads:h_dim + 2 * heads, :]
    bound = jnp.max(us, axis=1) + jnp.max(vd, axis=1)
    m = jnp.max(jnp.maximum(bound, NEG_SLOPE * bound))
    aux_row = h_dim + heads
    tab_ref[aux_row, 0:heads] = av[:, 2]
    tab_ref[aux_row, heads:2 * heads] = av[:, 3]
    tab_ref[aux_row, 2 * heads:2 * heads + L] = jnp.full((L,), m,
                                                         dtype=jnp.float32)


def _prologue(x, w, av):
    n = x.shape[0]
    npad = ((n + 127) // 128) * 128
    return pl.pallas_call(
        functools.partial(_prologue_body, npad),
        out_shape=jax.ShapeDtypeStruct((16, npad), jnp.float32),
    )(x, w, av)


def _sc_body(heads, epw, unroll, e_total, npad, *refs):
    hg_heads = heads // 2
    epad = e_total
    esd_ref, tab_ref, out_ref = refs[0:3]
    scr = refs[3:]
    es_v, ed_v, h0_v, h1_v = scr[0:4]
    us_vs = scr[4:4 + hg_heads]
    aux_v, o_v, sem = scr[4 + hg_heads:7 + hg_heads]

    cid = lax.axis_index("c")
    sid = lax.axis_index("s")
    wid = sid * NC + cid
    hg = wid % 2
    esl = wid // 2

    copies = [
        pltpu.async_copy(esd_ref.at[pl.ds(esl * epw, epw)], es_v, sem),
        pltpu.async_copy(esd_ref.at[pl.ds(epad + esl * epw, epw)], ed_v, sem),
        pltpu.async_copy(tab_ref.at[0], h0_v, sem),
        pltpu.async_copy(tab_ref.at[1], h1_v, sem),
        pltpu.async_copy(tab_ref.at[2 + heads, pl.ds(0, 4 * L)], aux_v, sem),
    ]

    @pl.when(hg == 0)
    def _():
        cs = [pltpu.async_copy(tab_ref.at[2 + h], us_vs[h], sem)
              for h in range(hg_heads)]
        for c in cs:
            c.wait()

    @pl.when(hg == 1)
    def _():
        cs = [pltpu.async_copy(tab_ref.at[2 + hg_heads + h], us_vs[h], sem)
              for h in range(hg_heads)]
        for c in cs:
            c.wait()

    for c in copies:
        c.wait()

    m_vec = aux_v[pl.ds(2 * heads, L)]
    hb = hg * hg_heads
    av2 = [plsc.load_gather(aux_v, [jnp.full((L,), h, jnp.int32) + hb])
           for h in range(hg_heads)]
    av3 = [plsc.load_gather(aux_v, [jnp.full((L,), heads + h, jnp.int32) + hb])
           for h in range(hg_heads)]

    init = tuple(jnp.zeros((L,), jnp.float32) for _ in range(3 * hg_heads))

    @plsc.parallel_loop(0, epw // L, 1, unroll=unroll, carry=init)
    def accs(i, accs):
        accs = list(accs)
        s = es_v[pl.ds(i * L, L)]
        d = ed_v[pl.ds(i * L, L)]
        hd0 = plsc.load_gather(h0_v, [d])
        hd1 = plsc.load_gather(h1_v, [d])
        for h in range(hg_heads):
            us_h = plsc.load_gather(us_vs[h], [s])
            z = us_h + av2[h] * hd0 + av3[h] * hd1
            y = jnp.maximum(z, NEG_SLOPE * z)
            p = jnp.exp(y - m_vec)
            accs[3 * h] = accs[3 * h] + p
            accs[3 * h + 1] = accs[3 * h + 1] + p * hd0
            accs[3 * h + 2] = accs[3 * h + 2] + p * hd1
        return tuple(accs)

    for h in range(hg_heads):
        o_v[pl.ds((0 * hg_heads + h) * L, L)] = accs[3 * h]
        o_v[pl.ds((1 * hg_heads + h) * L, L)] = accs[3 * h + 1]
        o_v[pl.ds((2 * hg_heads + h) * L, L)] = accs[3 * h + 2]
    nacc = 3 * hg_heads * L
    pltpu.sync_copy(o_v, out_ref.at[pl.ds(wid * nacc, nacc)])


def _sc_main(esd, tab, heads):
    e_total = esd.shape[0] // 2
    epw = e_total // (NW // 2)
    npad = tab.shape[1]
    unroll = 2
    assert epw % (L * unroll) == 0
    mesh = plsc.VectorSubcoreMesh(core_axis_name="c", subcore_axis_name="s",
                                  num_cores=NC, num_subcores=NS)
    nacc = 3 * (heads // 2) * L
    f = pl.kernel(
        functools.partial(_sc_body, heads, epw, unroll, e_total, npad),
        out_type=jax.ShapeDtypeStruct((NW * nacc,), jnp.float32),
        mesh=mesh,
        compiler_params=pltpu.CompilerParams(needs_layout_passes=False),
        scratch_types=(
            [pltpu.VMEM((epw,), jnp.int32)] * 2
            + [pltpu.VMEM((npad,), jnp.float32)] * (2 + heads // 2)
            + [pltpu.VMEM((4 * L,), jnp.float32),
               pltpu.VMEM((nacc,), jnp.float32),
               pltpu.SemaphoreType.DMA]
        ),
    )
    return f(esd, tab)


def _epilogue_body(heads, p_ref, o_ref):
    hh = heads // 2
    t = jnp.sum(jnp.sum(p_ref[...], axis=-1), axis=0)
    s = jnp.concatenate([t[0, 0:hh], t[1, 0:hh]])
    t0 = jnp.concatenate([t[0, hh:2 * hh], t[1, hh:2 * hh]])
    t1 = jnp.concatenate([t[0, 2 * hh:3 * hh], t[1, 2 * hh:3 * hh]])
    o_ref[...] = jax.nn.sigmoid(jnp.stack([t0 / s, t1 / s], axis=0))


def _epilogue(partials, heads):
    return pl.pallas_call(
        functools.partial(_epilogue_body, heads),
        out_shape=jax.ShapeDtypeStruct((2, heads), jnp.float32),
    )(partials)


def kernel(x, edges, a, W, attention_vectors):
    del a
    e = edges.shape[0]
    heads = attention_vectors.shape[0]
    assert e % (NW * L) == 0

    tab = _prologue(x, W, attention_vectors)
    esd = edges.astype(jnp.int32).T.reshape(-1)
    partials = _sc_main(esd, tab, heads)
    out28 = _epilogue(
        partials.reshape(NW // 2, 2, 3 * (heads // 2), L), heads)
    return out28.T
